# baseline (speedup 1.0000x reference)
import jax
import jax.numpy as jnp
import numpy as np
from jax import lax
from jax.experimental import pallas as pl
from jax.experimental.pallas import tpu as pltpu
from jax.experimental.pallas import tpu_sc as plsc

N = 10000
E = 320000
E2 = E // 2
STATE = 128
MSG = 64
PK = 2 * MSG
D_CUTOFF = 5.0
N_RBF = 10

NC = 2
NS = 16
NW = NC * NS
EPW = E // NW
CH = 1000
SUB = 125
KSUB = CH // SUB
CH2 = CH // 2
KSUB2 = CH2 // SUB
NCHUNK = EPW // CH
RPT = N // NS

BN = 2000
BE2 = 1280

_F32 = jnp.float32
_SQRT2 = np.sqrt(2.0).astype(np.float32)
_RBF_WIDTH = np.float32(D_CUTOFF / (N_RBF - 1))


def _mm(a, b):
    return lax.dot_general(a, b, (((1,), (0,)), ((), ())),
                           preferred_element_type=_F32,
                           precision=lax.Precision.DEFAULT)


def _gelu(x):
    return 0.5 * x * (1.0 + lax.erf(x / _SQRT2))


def _full(spec_shape):
    return pl.BlockSpec(spec_shape, lambda i: tuple(0 for _ in spec_shape))


def _rows(bs, cols):
    return pl.BlockSpec((bs, cols), lambda i: (i, 0))


def _blkdiag(w):
    k, m = w.shape
    z = jnp.zeros((k, m), _F32)
    return jnp.concatenate([jnp.concatenate([w, z], axis=1),
                            jnp.concatenate([z, w], axis=1)], axis=0)


def _embed_body(mi, e1, be1, e2, be2, wa, wb, h_ref, a_ref, b_ref):
    t = _gelu(_mm(mi[...], e1[...]) + be1[...])
    h = _mm(t, e2[...]) + be2[...]
    h_ref[...] = h
    a_ref[...] = _mm(h, wa[...])
    b_ref[...] = _mm(h, wb[...])


def _edge_feats(xs, xd, isb):
    dx = xs[...][:, 0:3] - xd[...][:, 0:3]
    d2 = jnp.sum(dx * dx, axis=1, keepdims=True)
    dist = jnp.sqrt(d2 + 1e-12)
    k = lax.broadcasted_iota(jnp.int32, (1, N_RBF), 1).astype(_F32)
    rbf = jnp.exp(-((dist * (1.0 / _RBF_WIDTH) - k) ** 2))
    return jnp.concatenate([isb[...], dist, rbf], axis=1)


def _edge0_body(pa, pb, xse, xso, xde, xdo, ibe, ibo,
                w1c, b1, w2, b2, w3, b3, msg_ref, efe_ref, efo_ref):
    efe = _edge_feats(xse, xde, ibe)
    efo = _edge_feats(xso, xdo, ibo)
    efe_ref[...] = efe
    efo_ref[...] = efo
    efw = jnp.concatenate([_mm(efe, w1c[...]), _mm(efo, w1c[...])], axis=1)
    t = _gelu(pa[...] + pb[...] + efw + b1[...])
    t = _gelu(_mm(t, w2[...]) + b2[...])
    msg_ref[...] = _mm(t, w3[...]) + b3[...]


def _edge1_body(pa, pb, efe, efo, w1c, b1, w2, b2, w3, b3, msg_ref):
    efw = jnp.concatenate([_mm(efe[...], w1c[...]),
                           _mm(efo[...], w1c[...])], axis=1)
    t = _gelu(pa[...] + pb[...] + efw + b1[...])
    t = _gelu(_mm(t, w2[...]) + b2[...])
    msg_ref[...] = _mm(t, w3[...]) + b3[...]


def _upd_body(h, p, u1a, u1b, b1, u2, b2, u3, b3, wa, wb,
              h_ref, a_ref, b_ref):
    nm = p[...][0] + p[...][1]
    t = _gelu(_mm(h[...], u1a[...]) + _mm(nm, u1b[...]) + b1[...])
    t = _gelu(_mm(t, u2[...]) + b2[...])
    hn = h[...] + _mm(t, u3[...]) + b3[...]
    h_ref[...] = hn
    a_ref[...] = _mm(hn, wa[...])
    b_ref[...] = _mm(hn, wb[...])


def _upd_last_body(h, p, u1a, u1b, b1, u2, b2, u3, b3, h_ref):
    nm = p[...][0] + p[...][1]
    t = _gelu(_mm(h[...], u1a[...]) + _mm(nm, u1b[...]) + b1[...])
    t = _gelu(_mm(t, u2[...]) + b2[...])
    h_ref[...] = h[...] + _mm(t, u3[...]) + b3[...]


def _mesh():
    return plsc.VectorSubcoreMesh(core_axis_name="c", subcore_axis_name="s",
                                  num_cores=NC, num_subcores=NS)


def _indirect_gather(table_hbm, idx2, nsub, buf, sem):
    for j in range(nsub):
        pltpu.async_copy(table_hbm.at[idx2.at[j]],
                         buf.at[pl.ds(j * SUB, SUB)], sem).wait()


def _gather0_body(a_hbm, b_hbm, xp_hbm, src_hbm, dst_hbm,
                  se_hbm, so_hbm, de_hbm, do_hbm,
                  prea_hbm, preb_hbm, xse_hbm, xso_hbm, xde_hbm, xdo_hbm,
                  idx_s, idx_d, idx4, rows, xbuf, sem):
    c = lax.axis_index("c")
    s = lax.axis_index("s")
    wid = s * NC + c

    def chunk(i, carry):
        base = wid * EPW + i * CH
        rbase = base // SUB
        pltpu.sync_copy(src_hbm.at[pl.ds(rbase, KSUB)], idx_s)
        pltpu.sync_copy(dst_hbm.at[pl.ds(rbase, KSUB)], idx_d)
        _indirect_gather(a_hbm, idx_s, KSUB, rows, sem)
        pltpu.sync_copy(rows, prea_hbm.at[pl.ds(base, CH)])
        _indirect_gather(b_hbm, idx_d, KSUB, rows, sem)
        pltpu.sync_copy(rows, preb_hbm.at[pl.ds(base, CH)])
        base2 = wid * (EPW // 2) + i * CH2
        rbase2 = base2 // SUB
        for ih, oh in ((se_hbm, xse_hbm), (so_hbm, xso_hbm),
                       (de_hbm, xde_hbm), (do_hbm, xdo_hbm)):
            pltpu.sync_copy(ih.at[pl.ds(rbase2, KSUB2)], idx4)
            _indirect_gather(xp_hbm, idx4, KSUB2, xbuf, sem)
            pltpu.sync_copy(xbuf, oh.at[pl.ds(base2, CH2)])
        return carry

    lax.fori_loop(0, NCHUNK, chunk, 0)


def _gather1_body(a_hbm, b_hbm, src_hbm, dst_hbm, prea_hbm, preb_hbm,
                  idx_s, idx_d, rows, sem):
    c = lax.axis_index("c")
    s = lax.axis_index("s")
    wid = s * NC + c

    def chunk(i, carry):
        base = wid * EPW + i * CH
        rbase = base // SUB
        pltpu.sync_copy(src_hbm.at[pl.ds(rbase, KSUB)], idx_s)
        pltpu.sync_copy(dst_hbm.at[pl.ds(rbase, KSUB)], idx_d)
        _indirect_gather(a_hbm, idx_s, KSUB, rows, sem)
        pltpu.sync_copy(rows, prea_hbm.at[pl.ds(base, CH)])
        _indirect_gather(b_hbm, idx_d, KSUB, rows, sem)
        pltpu.sync_copy(rows, preb_hbm.at[pl.ds(base, CH)])
        return carry

    lax.fori_loop(0, NCHUNK, chunk, 0)


def _scatter_body(msg_hbm, dst_hbm, zeros_hbm, out_hbm, idx_v, buf, acc, sem):
    c = lax.axis_index("c")
    s = lax.axis_index("s")
    r0 = s * RPT
    pltpu.sync_copy(zeros_hbm, acc.at[pl.ds(r0, RPT)])
    plsc.subcore_barrier()

    def chunk(i, carry):
        base = (c * NS + s) * EPW + i * CH
        pltpu.sync_copy(msg_hbm.at[pl.ds(base, CH)], buf)
        pltpu.sync_copy(dst_hbm.at[pl.ds(base // SUB, KSUB)], idx_v)
        for j in range(KSUB):
            pltpu.sync_copy(buf.at[pl.ds(j * SUB, SUB)],
                            acc.at[idx_v.at[j]], add=True)
        return carry

    lax.fori_loop(0, NCHUNK, chunk, 0)
    plsc.subcore_barrier()
    pltpu.sync_copy(acc.at[pl.ds(r0, RPT)], out_hbm.at[c, pl.ds(r0, RPT)])


def _sc_gather0(a, b, xpad, src, dst, se, so, de, do):
    return pl.kernel(
        _gather0_body,
        out_type=[
            jax.ShapeDtypeStruct((E, MSG), _F32),
            jax.ShapeDtypeStruct((E, MSG), _F32),
            jax.ShapeDtypeStruct((E2, 8), _F32),
            jax.ShapeDtypeStruct((E2, 8), _F32),
            jax.ShapeDtypeStruct((E2, 8), _F32),
            jax.ShapeDtypeStruct((E2, 8), _F32),
        ],
        mesh=_mesh(),
        compiler_params=pltpu.CompilerParams(use_tc_tiling_on_sc=False),
        scratch_types=[
            pltpu.VMEM((KSUB, SUB), jnp.int32),
            pltpu.VMEM((KSUB, SUB), jnp.int32),
            pltpu.VMEM((KSUB2, SUB), jnp.int32),
            pltpu.VMEM((CH, MSG), _F32),
            pltpu.VMEM((CH2, 8), _F32),
            pltpu.SemaphoreType.DMA,
        ],
    )(a, b, xpad, src, dst, se, so, de, do)


def _sc_gather1(a, b, src, dst):
    return pl.kernel(
        _gather1_body,
        out_type=[
            jax.ShapeDtypeStruct((E, MSG), _F32),
            jax.ShapeDtypeStruct((E, MSG), _F32),
        ],
        mesh=_mesh(),
        compiler_params=pltpu.CompilerParams(use_tc_tiling_on_sc=False),
        scratch_types=[
            pltpu.VMEM((KSUB, SUB), jnp.int32),
            pltpu.VMEM((KSUB, SUB), jnp.int32),
            pltpu.VMEM((CH, MSG), _F32),
            pltpu.SemaphoreType.DMA,
        ],
    )(a, b, src, dst)


def _sc_scatter(msg, dst, zeros_nm):
    return pl.kernel(
        _scatter_body,
        out_type=jax.ShapeDtypeStruct((NC, N, MSG), _F32),
        mesh=_mesh(),
        compiler_params=pltpu.CompilerParams(use_tc_tiling_on_sc=False),
        scratch_types=[
            pltpu.VMEM((KSUB, SUB), jnp.int32),
            pltpu.VMEM((CH, MSG), _F32),
            pltpu.VMEM_SHARED((N, MSG), _F32),
            pltpu.SemaphoreType.DMA,
        ],
    )(msg, dst, zeros_nm)


def _tc_embed(mi, e1, be1, e2, be2, wa, wb):
    return pl.pallas_call(
        _embed_body,
        grid=(N // BN,),
        in_specs=[_rows(BN, 32), _full((32, 64)), _full((1, 64)),
                  _full((64, STATE)), _full((1, STATE)),
                  _full((STATE, MSG)), _full((STATE, MSG))],
        out_specs=[_rows(BN, STATE), _rows(BN, MSG), _rows(BN, MSG)],
        out_shape=[jax.ShapeDtypeStruct((N, STATE), _F32),
                   jax.ShapeDtypeStruct((N, MSG), _F32),
                   jax.ShapeDtypeStruct((N, MSG), _F32)],
    )(mi, e1, be1, e2, be2, wa, wb)


def _tc_edge0(pa, pb, xse, xso, xde, xdo, ibe, ibo, w1c, b1, w2, b2, w3, b3):
    return pl.pallas_call(
        _edge0_body,
        grid=(E2 // BE2,),
        in_specs=[_rows(BE2, PK), _rows(BE2, PK),
                  _rows(BE2, 8), _rows(BE2, 8), _rows(BE2, 8), _rows(BE2, 8),
                  _rows(BE2, 1), _rows(BE2, 1),
                  _full((12, MSG)), _full((1, PK)),
                  _full((PK, PK)), _full((1, PK)),
                  _full((PK, PK)), _full((1, PK))],
        out_specs=[_rows(BE2, PK), _rows(BE2, 12), _rows(BE2, 12)],
        out_shape=[jax.ShapeDtypeStruct((E2, PK), _F32),
                   jax.ShapeDtypeStruct((E2, 12), _F32),
                   jax.ShapeDtypeStruct((E2, 12), _F32)],
    )(pa, pb, xse, xso, xde, xdo, ibe, ibo, w1c, b1, w2, b2, w3, b3)


def _tc_edge1(pa, pb, efe, efo, w1c, b1, w2, b2, w3, b3):
    return pl.pallas_call(
        _edge1_body,
        grid=(E2 // BE2,),
        in_specs=[_rows(BE2, PK), _rows(BE2, PK),
                  _rows(BE2, 12), _rows(BE2, 12),
                  _full((12, MSG)), _full((1, PK)),
                  _full((PK, PK)), _full((1, PK)),
                  _full((PK, PK)), _full((1, PK))],
        out_specs=_rows(BE2, PK),
        out_shape=jax.ShapeDtypeStruct((E2, PK), _F32),
    )(pa, pb, efe, efo, w1c, b1, w2, b2, w3, b3)


def _tc_update(h, p, u1a, u1b, b1, u2, b2, u3, b3, wa, wb):
    pspec = pl.BlockSpec((NC, BN, MSG), lambda i: (0, i, 0))
    return pl.pallas_call(
        _upd_body,
        grid=(N // BN,),
        in_specs=[_rows(BN, STATE), pspec,
                  _full((STATE, STATE)), _full((MSG, STATE)), _full((1, STATE)),
                  _full((STATE, STATE)), _full((1, STATE)),
                  _full((STATE, STATE)), _full((1, STATE)),
                  _full((STATE, MSG)), _full((STATE, MSG))],
        out_specs=[_rows(BN, STATE), _rows(BN, MSG), _rows(BN, MSG)],
        out_shape=[jax.ShapeDtypeStruct((N, STATE), _F32),
                   jax.ShapeDtypeStruct((N, MSG), _F32),
                   jax.ShapeDtypeStruct((N, MSG), _F32)],
    )(h, p, u1a, u1b, b1, u2, b2, u3, b3, wa, wb)


def _tc_update_last(h, p, u1a, u1b, b1, u2, b2, u3, b3):
    pspec = pl.BlockSpec((NC, BN, MSG), lambda i: (0, i, 0))
    return pl.pallas_call(
        _upd_last_body,
        grid=(N // BN,),
        in_specs=[_rows(BN, STATE), pspec,
                  _full((STATE, STATE)), _full((MSG, STATE)), _full((1, STATE)),
                  _full((STATE, STATE)), _full((1, STATE)),
                  _full((STATE, STATE)), _full((1, STATE))],
        out_specs=_rows(BN, STATE),
        out_shape=jax.ShapeDtypeStruct((N, STATE), _F32),
    )(h, p, u1a, u1b, b1, u2, b2, u3, b3)


def _pk(x):
    return jnp.reshape(x, (E2, PK))


def kernel(molecule_info, x, edge_index, is_bond, params):
    src = edge_index[0].reshape(E // SUB, SUB)
    dst = edge_index[1].reshape(E // SUB, SUB)
    eit0 = edge_index[0].reshape(E2, 2).T
    eit1 = edge_index[1].reshape(E2, 2).T
    se = eit0[0].reshape(E2 // SUB, SUB)
    so = eit0[1].reshape(E2 // SUB, SUB)
    de = eit1[0].reshape(E2 // SUB, SUB)
    do = eit1[1].reshape(E2 // SUB, SUB)
    ibt = is_bond.reshape(E2, 2).T
    ibe = ibt[0][:, None]
    ibo = ibt[1][:, None]
    xpad = jnp.pad(x, ((0, 0), (0, 5)))

    (e1, be1), (e2, be2) = params["embed"]
    msg_w = []
    for l in range(2):
        (w1, b1), (w2, b2), (w3, b3) = params["msg"][l]
        msg_w.append((w1[:STATE], w1[STATE:2 * STATE], w1[2 * STATE:],
                      jnp.tile(b1[None, :], (1, 2)),
                      _blkdiag(w2), jnp.tile(b2[None, :], (1, 2)),
                      _blkdiag(w3), jnp.tile(b3[None, :], (1, 2))))
    upd_w = []
    for l in range(2):
        (u1, ub1), (u2, ub2), (u3, ub3) = params["upd"][l]
        upd_w.append((u1[:STATE], u1[STATE:], ub1[None, :],
                      u2, ub2[None, :], u3, ub3[None, :]))

    zeros_nm = jnp.zeros((RPT, MSG), _F32)

    w1a0, w1b0, w1c0, mb10, mw20, mb20, mw30, mb30 = msg_w[0]
    w1a1, w1b1, w1c1, mb11, mw21, mb21, mw31, mb31 = msg_w[1]

    h0, a0, b0 = _tc_embed(molecule_info, e1, be1[None, :], e2, be2[None, :],
                           w1a0, w1b0)
    pa0, pb0, xse, xso, xde, xdo = _sc_gather0(a0, b0, xpad, src, dst,
                                               se, so, de, do)
    msg0, efe, efo = _tc_edge0(_pk(pa0), _pk(pb0), xse, xso, xde, xdo,
                               ibe, ibo, w1c0, mb10, mw20, mb20, mw30, mb30)
    p0 = _sc_scatter(jnp.reshape(msg0, (E, MSG)), dst, zeros_nm)
    u1a, u1b, ub1, u2, ub2, u3, ub3 = upd_w[0]
    h1, a1, b1v = _tc_update(h0, p0, u1a, u1b, ub1, u2, ub2, u3, ub3,
                             w1a1, w1b1)

    pa1, pb1 = _sc_gather1(a1, b1v, src, dst)
    msg1 = _tc_edge1(_pk(pa1), _pk(pb1), efe, efo,
                     w1c1, mb11, mw21, mb21, mw31, mb31)
    p1 = _sc_scatter(jnp.reshape(msg1, (E, MSG)), dst, zeros_nm)
    u1a, u1b, ub1, u2, ub2, u3, ub3 = upd_w[1]
    h2 = _tc_update_last(h1, p1, u1a, u1b, ub1, u2, ub2, u3, ub3)
    return h2


# speedup vs baseline: 1.1408x; 1.1408x over previous
import jax
import jax.numpy as jnp
import numpy as np
from jax import lax
from jax.experimental import pallas as pl
from jax.experimental.pallas import tpu as pltpu
from jax.experimental.pallas import tpu_sc as plsc

N = 10000
E = 320000
E2 = E // 2
STATE = 128
MSG = 64
PK = 2 * MSG
D_CUTOFF = 5.0
N_RBF = 10

NC = 2
NS = 16
NW = NC * NS
EPW = E // NW
CH = 1000
SUB = 125
KSUB = CH // SUB
CH2 = CH // 2
KSUB2 = CH2 // SUB
NCHUNK = EPW // CH
RPT = N // NS

BN = 2000
BE2 = 1280

_F32 = jnp.float32
_SQRT2 = np.sqrt(2.0).astype(np.float32)
_RBF_WIDTH = np.float32(D_CUTOFF / (N_RBF - 1))


def _mm(a, b):
    return lax.dot_general(a, b, (((1,), (0,)), ((), ())),
                           preferred_element_type=_F32,
                           precision=lax.Precision.DEFAULT)


def _gelu(x):
    return 0.5 * x * (1.0 + lax.erf(x / _SQRT2))


def _full(spec_shape):
    return pl.BlockSpec(spec_shape, lambda i: tuple(0 for _ in spec_shape))


def _rows(bs, cols):
    return pl.BlockSpec((bs, cols), lambda i: (i, 0))


def _blkdiag(w):
    k, m = w.shape
    z = jnp.zeros((k, m), _F32)
    return jnp.concatenate([jnp.concatenate([w, z], axis=1),
                            jnp.concatenate([z, w], axis=1)], axis=0)


def _embed_body(mi, e1, be1, e2, be2, wa, wb, h_ref, a_ref, b_ref):
    t = _gelu(_mm(mi[...], e1[...]) + be1[...])
    h = _mm(t, e2[...]) + be2[...]
    h_ref[...] = h
    a_ref[...] = _mm(h, wa[...])
    b_ref[...] = _mm(h, wb[...])


def _edge_feats(xs, xd, isb):
    dx = xs[...][:, 0:3] - xd[...][:, 0:3]
    d2 = jnp.sum(dx * dx, axis=1, keepdims=True)
    dist = jnp.sqrt(d2 + 1e-12)
    k = lax.broadcasted_iota(jnp.int32, (1, N_RBF), 1).astype(_F32)
    rbf = jnp.exp(-((dist * (1.0 / _RBF_WIDTH) - k) ** 2))
    return jnp.concatenate([isb[...], dist, rbf], axis=1)


def _edge0_body(pa, pb, xse, xso, xde, xdo, ibe, ibo,
                w1c, b1, w2, b2, w3, b3, msg_ref, efe_ref, efo_ref):
    efe = _edge_feats(xse, xde, ibe)
    efo = _edge_feats(xso, xdo, ibo)
    efe_ref[...] = efe
    efo_ref[...] = efo
    efw = jnp.concatenate([_mm(efe, w1c[...]), _mm(efo, w1c[...])], axis=1)
    t = _gelu(pa[...] + pb[...] + efw + b1[...])
    t = _gelu(_mm(t, w2[...]) + b2[...])
    msg_ref[...] = _mm(t, w3[...]) + b3[...]


def _edge1_body(pa, pb, efe, efo, w1c, b1, w2, b2, w3, b3, msg_ref):
    efw = jnp.concatenate([_mm(efe[...], w1c[...]),
                           _mm(efo[...], w1c[...])], axis=1)
    t = _gelu(pa[...] + pb[...] + efw + b1[...])
    t = _gelu(_mm(t, w2[...]) + b2[...])
    msg_ref[...] = _mm(t, w3[...]) + b3[...]


def _upd_body(h, p, u1a, u1b, b1, u2, b2, u3, b3, wa, wb,
              h_ref, a_ref, b_ref):
    nm = p[...][0] + p[...][1]
    t = _gelu(_mm(h[...], u1a[...]) + _mm(nm, u1b[...]) + b1[...])
    t = _gelu(_mm(t, u2[...]) + b2[...])
    hn = h[...] + _mm(t, u3[...]) + b3[...]
    h_ref[...] = hn
    a_ref[...] = _mm(hn, wa[...])
    b_ref[...] = _mm(hn, wb[...])


def _upd_last_body(h, p, u1a, u1b, b1, u2, b2, u3, b3, h_ref):
    nm = p[...][0] + p[...][1]
    t = _gelu(_mm(h[...], u1a[...]) + _mm(nm, u1b[...]) + b1[...])
    t = _gelu(_mm(t, u2[...]) + b2[...])
    h_ref[...] = h[...] + _mm(t, u3[...]) + b3[...]


def _mesh():
    return plsc.VectorSubcoreMesh(core_axis_name="c", subcore_axis_name="s",
                                  num_cores=NC, num_subcores=NS)


def _indirect_gather(table_hbm, idx2, nsub, buf, sem):
    for j in range(nsub):
        pltpu.async_copy(table_hbm.at[idx2.at[j]],
                         buf.at[pl.ds(j * SUB, SUB)], sem).wait()


def _gather0_body(a_hbm, b_hbm, xp_hbm, src_hbm, dst_hbm,
                  se_hbm, so_hbm, de_hbm, do_hbm,
                  prea_hbm, preb_hbm, xse_hbm, xso_hbm, xde_hbm, xdo_hbm,
                  idx_s, idx_d, idx4, rows, xbuf, sem):
    c = lax.axis_index("c")
    s = lax.axis_index("s")
    wid = s * NC + c

    def chunk(i, carry):
        base = wid * EPW + i * CH
        rbase = base // SUB
        pltpu.sync_copy(src_hbm.at[pl.ds(rbase, KSUB)], idx_s)
        pltpu.sync_copy(dst_hbm.at[pl.ds(rbase, KSUB)], idx_d)
        _indirect_gather(a_hbm, idx_s, KSUB, rows, sem)
        pltpu.sync_copy(rows, prea_hbm.at[pl.ds(base, CH)])
        _indirect_gather(b_hbm, idx_d, KSUB, rows, sem)
        pltpu.sync_copy(rows, preb_hbm.at[pl.ds(base, CH)])
        base2 = wid * (EPW // 2) + i * CH2
        rbase2 = base2 // SUB
        for ih, oh in ((se_hbm, xse_hbm), (so_hbm, xso_hbm),
                       (de_hbm, xde_hbm), (do_hbm, xdo_hbm)):
            pltpu.sync_copy(ih.at[pl.ds(rbase2, KSUB2)], idx4)
            _indirect_gather(xp_hbm, idx4, KSUB2, xbuf, sem)
            pltpu.sync_copy(xbuf, oh.at[pl.ds(base2, CH2)])
        return carry

    lax.fori_loop(0, NCHUNK, chunk, 0)


def _gather1_body(a_hbm, b_hbm, src_hbm, dst_hbm, prea_hbm, preb_hbm,
                  idx_s, idx_d, rows, sem):
    c = lax.axis_index("c")
    s = lax.axis_index("s")
    wid = s * NC + c

    def chunk(i, carry):
        base = wid * EPW + i * CH
        rbase = base // SUB
        pltpu.sync_copy(src_hbm.at[pl.ds(rbase, KSUB)], idx_s)
        pltpu.sync_copy(dst_hbm.at[pl.ds(rbase, KSUB)], idx_d)
        _indirect_gather(a_hbm, idx_s, KSUB, rows, sem)
        pltpu.sync_copy(rows, prea_hbm.at[pl.ds(base, CH)])
        _indirect_gather(b_hbm, idx_d, KSUB, rows, sem)
        pltpu.sync_copy(rows, preb_hbm.at[pl.ds(base, CH)])
        return carry

    lax.fori_loop(0, NCHUNK, chunk, 0)


def _scatter_body(msg_hbm, dst_hbm, zeros_hbm, out_hbm, idx_v, buf, acc, sem):
    c = lax.axis_index("c")
    s = lax.axis_index("s")
    r0 = s * RPT
    pltpu.sync_copy(zeros_hbm, acc.at[pl.ds(r0, RPT)])
    plsc.subcore_barrier()

    def chunk(i, carry):
        base = (c * NS + s) * EPW + i * CH
        pltpu.sync_copy(msg_hbm.at[pl.ds(base, CH)], buf)
        pltpu.sync_copy(dst_hbm.at[pl.ds(base // SUB, KSUB)], idx_v)
        for j in range(KSUB):
            pltpu.sync_copy(buf.at[pl.ds(j * SUB, SUB)],
                            acc.at[idx_v.at[j]], add=True)
        return carry

    lax.fori_loop(0, NCHUNK, chunk, 0)
    plsc.subcore_barrier()
    pltpu.sync_copy(acc.at[pl.ds(r0, RPT)], out_hbm.at[c, pl.ds(r0, RPT)])


def _sc_gather0(a, b, xpad, src, dst, se, so, de, do):
    return pl.kernel(
        _gather0_body,
        out_type=[
            jax.ShapeDtypeStruct((E, MSG), _F32),
            jax.ShapeDtypeStruct((E, MSG), _F32),
            jax.ShapeDtypeStruct((E2, 8), _F32),
            jax.ShapeDtypeStruct((E2, 8), _F32),
            jax.ShapeDtypeStruct((E2, 8), _F32),
            jax.ShapeDtypeStruct((E2, 8), _F32),
        ],
        mesh=_mesh(),
        compiler_params=pltpu.CompilerParams(use_tc_tiling_on_sc=False),
        scratch_types=[
            pltpu.VMEM((KSUB, SUB), jnp.int32),
            pltpu.VMEM((KSUB, SUB), jnp.int32),
            pltpu.VMEM((KSUB2, SUB), jnp.int32),
            pltpu.VMEM((CH, MSG), _F32),
            pltpu.VMEM((CH2, 8), _F32),
            pltpu.SemaphoreType.DMA,
        ],
    )(a, b, xpad, src, dst, se, so, de, do)


def _sc_gather1(a, b, src, dst):
    return pl.kernel(
        _gather1_body,
        out_type=[
            jax.ShapeDtypeStruct((E, MSG), _F32),
            jax.ShapeDtypeStruct((E, MSG), _F32),
        ],
        mesh=_mesh(),
        compiler_params=pltpu.CompilerParams(use_tc_tiling_on_sc=False),
        scratch_types=[
            pltpu.VMEM((KSUB, SUB), jnp.int32),
            pltpu.VMEM((KSUB, SUB), jnp.int32),
            pltpu.VMEM((CH, MSG), _F32),
            pltpu.SemaphoreType.DMA,
        ],
    )(a, b, src, dst)


def _sc_scatter(msg, dst, zeros_nm):
    return pl.kernel(
        _scatter_body,
        out_type=jax.ShapeDtypeStruct((NC, N, MSG), _F32),
        mesh=_mesh(),
        compiler_params=pltpu.CompilerParams(use_tc_tiling_on_sc=False),
        scratch_types=[
            pltpu.VMEM((KSUB, SUB), jnp.int32),
            pltpu.VMEM((CH, MSG), _F32),
            pltpu.VMEM_SHARED((N, MSG), _F32),
            pltpu.SemaphoreType.DMA,
        ],
    )(msg, dst, zeros_nm)


def _tc_embed(mi, e1, be1, e2, be2, wa, wb):
    return pl.pallas_call(
        _embed_body,
        grid=(N // BN,),
        in_specs=[_rows(BN, 32), _full((32, 64)), _full((1, 64)),
                  _full((64, STATE)), _full((1, STATE)),
                  _full((STATE, MSG)), _full((STATE, MSG))],
        out_specs=[_rows(BN, STATE), _rows(BN, MSG), _rows(BN, MSG)],
        out_shape=[jax.ShapeDtypeStruct((N, STATE), _F32),
                   jax.ShapeDtypeStruct((N, MSG), _F32),
                   jax.ShapeDtypeStruct((N, MSG), _F32)],
    )(mi, e1, be1, e2, be2, wa, wb)


def _tc_edge0(pa, pb, xse, xso, xde, xdo, ibe, ibo, w1c, b1, w2, b2, w3, b3):
    return pl.pallas_call(
        _edge0_body,
        grid=(E2 // BE2,),
        in_specs=[_rows(BE2, PK), _rows(BE2, PK),
                  _rows(BE2, 8), _rows(BE2, 8), _rows(BE2, 8), _rows(BE2, 8),
                  _rows(BE2, 1), _rows(BE2, 1),
                  _full((12, MSG)), _full((1, PK)),
                  _full((PK, PK)), _full((1, PK)),
                  _full((PK, PK)), _full((1, PK))],
        out_specs=[_rows(BE2, PK), _rows(BE2, 12), _rows(BE2, 12)],
        out_shape=[jax.ShapeDtypeStruct((E2, PK), _F32),
                   jax.ShapeDtypeStruct((E2, 12), _F32),
                   jax.ShapeDtypeStruct((E2, 12), _F32)],
    )(pa, pb, xse, xso, xde, xdo, ibe, ibo, w1c, b1, w2, b2, w3, b3)


def _tc_edge1(pa, pb, efe, efo, w1c, b1, w2, b2, w3, b3):
    return pl.pallas_call(
        _edge1_body,
        grid=(E2 // BE2,),
        in_specs=[_rows(BE2, PK), _rows(BE2, PK),
                  _rows(BE2, 12), _rows(BE2, 12),
                  _full((12, MSG)), _full((1, PK)),
                  _full((PK, PK)), _full((1, PK)),
                  _full((PK, PK)), _full((1, PK))],
        out_specs=_rows(BE2, PK),
        out_shape=jax.ShapeDtypeStruct((E2, PK), _F32),
    )(pa, pb, efe, efo, w1c, b1, w2, b2, w3, b3)


def _tc_update(h, p, u1a, u1b, b1, u2, b2, u3, b3, wa, wb):
    pspec = pl.BlockSpec((NC, BN, MSG), lambda i: (0, i, 0))
    return pl.pallas_call(
        _upd_body,
        grid=(N // BN,),
        in_specs=[_rows(BN, STATE), pspec,
                  _full((STATE, STATE)), _full((MSG, STATE)), _full((1, STATE)),
                  _full((STATE, STATE)), _full((1, STATE)),
                  _full((STATE, STATE)), _full((1, STATE)),
                  _full((STATE, MSG)), _full((STATE, MSG))],
        out_specs=[_rows(BN, STATE), _rows(BN, MSG), _rows(BN, MSG)],
        out_shape=[jax.ShapeDtypeStruct((N, STATE), _F32),
                   jax.ShapeDtypeStruct((N, MSG), _F32),
                   jax.ShapeDtypeStruct((N, MSG), _F32)],
    )(h, p, u1a, u1b, b1, u2, b2, u3, b3, wa, wb)


def _tc_update_last(h, p, u1a, u1b, b1, u2, b2, u3, b3):
    pspec = pl.BlockSpec((NC, BN, MSG), lambda i: (0, i, 0))
    return pl.pallas_call(
        _upd_last_body,
        grid=(N // BN,),
        in_specs=[_rows(BN, STATE), pspec,
                  _full((STATE, STATE)), _full((MSG, STATE)), _full((1, STATE)),
                  _full((STATE, STATE)), _full((1, STATE)),
                  _full((STATE, STATE)), _full((1, STATE))],
        out_specs=_rows(BN, STATE),
        out_shape=jax.ShapeDtypeStruct((N, STATE), _F32),
    )(h, p, u1a, u1b, b1, u2, b2, u3, b3)


def _pk(x):
    return jnp.reshape(x, (E2, PK))


def kernel(molecule_info, x, edge_index, is_bond, params):
    src = edge_index[0].reshape(E // SUB, SUB)
    dst = edge_index[1].reshape(E // SUB, SUB)
    se = edge_index[0, 0::2].reshape(E2 // SUB, SUB)
    so = edge_index[0, 1::2].reshape(E2 // SUB, SUB)
    de = edge_index[1, 0::2].reshape(E2 // SUB, SUB)
    do = edge_index[1, 1::2].reshape(E2 // SUB, SUB)
    ibe = is_bond[0::2][:, None]
    ibo = is_bond[1::2][:, None]
    xpad = jnp.pad(x, ((0, 0), (0, 5)))

    (e1, be1), (e2, be2) = params["embed"]
    msg_w = []
    for l in range(2):
        (w1, b1), (w2, b2), (w3, b3) = params["msg"][l]
        msg_w.append((w1[:STATE], w1[STATE:2 * STATE], w1[2 * STATE:],
                      jnp.tile(b1[None, :], (1, 2)),
                      _blkdiag(w2), jnp.tile(b2[None, :], (1, 2)),
                      _blkdiag(w3), jnp.tile(b3[None, :], (1, 2))))
    upd_w = []
    for l in range(2):
        (u1, ub1), (u2, ub2), (u3, ub3) = params["upd"][l]
        upd_w.append((u1[:STATE], u1[STATE:], ub1[None, :],
                      u2, ub2[None, :], u3, ub3[None, :]))

    zeros_nm = jnp.zeros((RPT, MSG), _F32)

    w1a0, w1b0, w1c0, mb10, mw20, mb20, mw30, mb30 = msg_w[0]
    w1a1, w1b1, w1c1, mb11, mw21, mb21, mw31, mb31 = msg_w[1]

    h0, a0, b0 = _tc_embed(molecule_info, e1, be1[None, :], e2, be2[None, :],
                           w1a0, w1b0)
    pa0, pb0, xse, xso, xde, xdo = _sc_gather0(a0, b0, xpad, src, dst,
                                               se, so, de, do)
    msg0, efe, efo = _tc_edge0(_pk(pa0), _pk(pb0), xse, xso, xde, xdo,
                               ibe, ibo, w1c0, mb10, mw20, mb20, mw30, mb30)
    p0 = _sc_scatter(jnp.reshape(msg0, (E, MSG)), dst, zeros_nm)
    u1a, u1b, ub1, u2, ub2, u3, ub3 = upd_w[0]
    h1, a1, b1v = _tc_update(h0, p0, u1a, u1b, ub1, u2, ub2, u3, ub3,
                             w1a1, w1b1)

    pa1, pb1 = _sc_gather1(a1, b1v, src, dst)
    msg1 = _tc_edge1(_pk(pa1), _pk(pb1), efe, efo,
                     w1c1, mb11, mw21, mb21, mw31, mb31)
    p1 = _sc_scatter(jnp.reshape(msg1, (E, MSG)), dst, zeros_nm)
    u1a, u1b, ub1, u2, ub2, u3, ub3 = upd_w[1]
    h2 = _tc_update_last(h1, p1, u1a, u1b, ub1, u2, ub2, u3, ub3)
    return h2
